# baseline (speedup 1.0000x reference)
import functools

import jax
import jax.numpy as jnp
from jax import lax
from jax.experimental import pallas as pl
from jax.experimental.pallas import tpu as pltpu
from jax.experimental.pallas import tpu_sc as plsc

N_NODES = 10000
N_EDGES = 320000
D_FEAT = 128
HIDDEN = 16

NC = 2
NS = 16
LANES = 16

CHUNK = 128
N_CHUNKS = N_EDGES // CHUNK
NB = N_CHUNKS // (NC * NS)
NBUF = 6
NGRP = NB // NBUF
EXTRA_CHUNKS = N_CHUNKS - NB * NC * NS
ROWS_PER_SUB = 624
ROWS_TAIL = N_NODES - NS * ROWS_PER_SUB

N_WIDE = N_NODES * HIDDEN // 128
N_WIDE_PAD = 1256
N_NODES_PAD = N_WIDE_PAD * 128 // HIDDEN

_sc_mesh = plsc.VectorSubcoreMesh(
    core_axis_name="c", subcore_axis_name="s", num_cores=NC, num_subcores=NS
)

_sc_params = pltpu.CompilerParams(use_tc_tiling_on_sc=False)


@functools.partial(
    pl.kernel,
    out_type=jax.ShapeDtypeStruct((NC, N_NODES_PAD, HIDDEN), jnp.float32),
    mesh=_sc_mesh,
    compiler_params=_sc_params,
    scratch_types=[
        pltpu.VMEM((NB + 1, CHUNK), jnp.int32),
        pltpu.VMEM((CHUNK, HIDDEN), jnp.float32),
        pltpu.VMEM((ROWS_PER_SUB, HIDDEN), jnp.float32),
        pltpu.VMEM_SHARED((N_NODES, HIDDEN), jnp.float32),
        pltpu.SemaphoreType.DMA,
    ],
)
def _hist(ei_hbm, out_hbm, idx_v, ones_v, zrows_v, acc_sh, sem):
    c = lax.axis_index("c")
    s = lax.axis_index("s")
    wid = s * NC + c

    @pl.loop(0, CHUNK)
    def _(i):
        ones_v[i, :] = jnp.ones((LANES,), jnp.float32)

    @pl.loop(0, ROWS_PER_SUB)
    def _(i):
        zrows_v[i, :] = jnp.zeros((LANES,), jnp.float32)

    pltpu.sync_copy(ei_hbm.at[1, pl.ds(wid * NB, NB)], idx_v.at[pl.ds(0, NB)])

    @pl.when(wid < EXTRA_CHUNKS)
    def _():
        pltpu.sync_copy(ei_hbm.at[1, pl.ds(NC * NS * NB + wid, 1)],
                        idx_v.at[pl.ds(NB, 1)])

    pltpu.sync_copy(zrows_v, acc_sh.at[pl.ds(s * ROWS_PER_SUB, ROWS_PER_SUB)])

    @pl.when(s == NS - 1)
    def _():
        pltpu.sync_copy(zrows_v.at[pl.ds(0, ROWS_TAIL)],
                        acc_sh.at[pl.ds(NS * ROWS_PER_SUB, ROWS_TAIL)])

    plsc.subcore_barrier()

    @pl.loop(0, NB)
    def _(j):
        pltpu.async_copy(ones_v, acc_sh.at[idx_v.at[j]], sem, add=True)

    @pl.when(wid < EXTRA_CHUNKS)
    def _():
        pltpu.async_copy(ones_v, acc_sh.at[idx_v.at[NB]], sem, add=True)

    @pl.loop(0, NB)
    def _(j):
        pltpu.make_async_copy(ones_v, acc_sh.at[idx_v.at[j]], sem).wait()

    @pl.when(wid < EXTRA_CHUNKS)
    def _():
        pltpu.make_async_copy(ones_v, acc_sh.at[idx_v.at[NB]], sem).wait()

    plsc.subcore_barrier()
    pltpu.sync_copy(
        acc_sh.at[pl.ds(s * ROWS_PER_SUB, ROWS_PER_SUB)],
        out_hbm.at[c, pl.ds(s * ROWS_PER_SUB, ROWS_PER_SUB)],
    )

    @pl.when(s == NS - 1)
    def _():
        pltpu.sync_copy(
            acc_sh.at[pl.ds(NS * ROWS_PER_SUB, ROWS_TAIL)],
            out_hbm.at[c, pl.ds(NS * ROWS_PER_SUB, ROWS_TAIL)],
        )


@functools.partial(
    pl.kernel,
    out_type=jax.ShapeDtypeStruct((NC, N_NODES_PAD, HIDDEN), jnp.float32),
    mesh=_sc_mesh,
    compiler_params=_sc_params,
    scratch_types=[
        pltpu.VMEM((NB + 1, CHUNK), jnp.int32),
        pltpu.VMEM((NB + 1, CHUNK), jnp.int32),
        [pltpu.VMEM((CHUNK, HIDDEN), jnp.float32) for _ in range(NBUF)],
        pltpu.VMEM((ROWS_PER_SUB, HIDDEN), jnp.float32),
        pltpu.VMEM_SHARED((N_NODES, HIDDEN), jnp.float32),
        [pltpu.SemaphoreType.DMA for _ in range(NBUF)],
    ],
)
def _segsum(ei_hbm, y_hbm, out_hbm,
            idx_s_v, idx_d_v, rows_bufs, zrows_v, acc_sh, sems):
    c = lax.axis_index("c")
    s = lax.axis_index("s")
    wid = s * NC + c

    @pl.loop(0, ROWS_PER_SUB)
    def _(i):
        zrows_v[i, :] = jnp.zeros((LANES,), jnp.float32)

    pltpu.sync_copy(ei_hbm.at[0, pl.ds(wid * NB, NB)], idx_s_v.at[pl.ds(0, NB)])
    pltpu.sync_copy(ei_hbm.at[1, pl.ds(wid * NB, NB)], idx_d_v.at[pl.ds(0, NB)])

    @pl.when(wid < EXTRA_CHUNKS)
    def _():
        pltpu.sync_copy(ei_hbm.at[0, pl.ds(NC * NS * NB + wid, 1)],
                        idx_s_v.at[pl.ds(NB, 1)])
        pltpu.sync_copy(ei_hbm.at[1, pl.ds(NC * NS * NB + wid, 1)],
                        idx_d_v.at[pl.ds(NB, 1)])

    rows_slc = pl.ds(s * ROWS_PER_SUB, ROWS_PER_SUB)
    tail_slc = pl.ds(NS * ROWS_PER_SUB, ROWS_TAIL)
    pltpu.sync_copy(zrows_v, acc_sh.at[rows_slc])

    @pl.when(s == NS - 1)
    def _():
        pltpu.sync_copy(zrows_v.at[pl.ds(0, ROWS_TAIL)], acc_sh.at[tail_slc])

    plsc.subcore_barrier()

    def gath(j, buf, sem):
        pltpu.async_copy(y_hbm.at[idx_s_v.at[j]], buf, sem)

    def wait_one(buf, sem):
        pltpu.make_async_copy(y_hbm.at[idx_s_v.at[0]], buf, sem).wait()

    def scat(j, buf, sem):
        pltpu.async_copy(buf, acc_sh.at[idx_d_v.at[j]], sem, add=True)

    for b in range(NBUF):
        gath(b, rows_bufs[b], sems[b])

    @pl.loop(0, NGRP)
    def _(g):
        j0 = g * NBUF
        for b in range(NBUF):
            wait_one(rows_bufs[b], sems[b])
            scat(j0 + b, rows_bufs[b], sems[b])

        @pl.when(g < NGRP - 1)
        def _():
            for b in range(NBUF):
                wait_one(rows_bufs[b], sems[b])
                gath(j0 + NBUF + b, rows_bufs[b], sems[b])

    for b in range(NBUF):
        wait_one(rows_bufs[b], sems[b])

    @pl.when(wid < EXTRA_CHUNKS)
    def _():
        gath(NB, rows_bufs[0], sems[0])
        wait_one(rows_bufs[0], sems[0])
        scat(NB, rows_bufs[0], sems[0])
        wait_one(rows_bufs[0], sems[0])

    plsc.subcore_barrier()
    pltpu.sync_copy(acc_sh.at[rows_slc], out_hbm.at[c, rows_slc])

    @pl.when(s == NS - 1)
    def _():
        pltpu.sync_copy(acc_sh.at[tail_slc], out_hbm.at[c, tail_slc])


def _blockdiag_mask(n_rep, blk_r, blk_c, dtype=jnp.float32):
    shape = (n_rep * blk_r, n_rep * blk_c)
    r = lax.broadcasted_iota(jnp.int32, shape, 0) // blk_r
    c = lax.broadcasted_iota(jnp.int32, shape, 1) // blk_c
    return (r == c).astype(dtype)


def _matmul_body(xw_ref, w_ref, y_ref):
    w = w_ref[...]
    wc = jnp.concatenate([w] * 8, axis=0)
    wt = jnp.concatenate([wc] * 8, axis=1)
    w2 = wt * _blockdiag_mask(8, D_FEAT, HIDDEN)
    y_ref[...] = jnp.dot(xw_ref[...], w2, preferred_element_type=jnp.float32)


def _matmul(x_w, w):
    return pl.pallas_call(
        _matmul_body,
        grid=(1,),
        in_specs=[
            pl.BlockSpec((N_WIDE, 8 * D_FEAT), lambda i: (0, 0)),
            pl.BlockSpec((D_FEAT, HIDDEN), lambda i: (0, 0)),
        ],
        out_specs=pl.BlockSpec((N_WIDE, 128), lambda i: (0, 0)),
        out_shape=jax.ShapeDtypeStruct((N_WIDE, 128), jnp.float32),
    )(x_w, w)


def _scale_body(xw_ref, degp_ref, y_ref):
    degp = degp_ref[:, :N_WIDE, :]
    deg = degp[0] + degp[1] + 1.0
    y_ref[...] = xw_ref[...] * lax.rsqrt(deg)


def _scale(xw_w, degp_w):
    return pl.pallas_call(
        _scale_body,
        grid=(1,),
        in_specs=[
            pl.BlockSpec((N_WIDE, 128), lambda i: (0, 0)),
            pl.BlockSpec((NC, N_WIDE_PAD, 128), lambda i: (0, 0, 0)),
        ],
        out_specs=pl.BlockSpec((N_WIDE, 128), lambda i: (0, 0)),
        out_shape=jax.ShapeDtypeStruct((N_WIDE, 128), jnp.float32),
    )(xw_w, degp_w)


def _final_body(sp_ref, y_ref, degp_ref, b_ref, o_ref):
    degp = degp_ref[:, :N_WIDE, :]
    sp = sp_ref[:, :N_WIDE, :]
    deg = degp[0] + degp[1] + 1.0
    dis = lax.rsqrt(deg)
    b_w = jnp.concatenate([b_ref[...]] * 8, axis=1)
    h = dis * (sp[0] + sp[1] + y_ref[...]) + b_w
    h = jnp.maximum(h, 0.0)
    e = jnp.exp(h)
    g = _blockdiag_mask(8, HIDDEN, HIDDEN)
    s = jnp.dot(e, g, preferred_element_type=jnp.float32)
    ls = h - jnp.log(s)
    i_id = lax.broadcasted_iota(jnp.int32, (128, HIDDEN), 0)
    j_id = lax.broadcasted_iota(jnp.int32, (128, HIDDEN), 1)
    parts = []
    for k in range(8):
        sel = (i_id == j_id + HIDDEN * k).astype(jnp.float32)
        nk = jnp.dot(ls, sel, preferred_element_type=jnp.float32)
        parts.append(nk.reshape(N_WIDE, 1, HIDDEN))
    o_ref[...] = jnp.concatenate(parts, axis=1).reshape(N_NODES, HIDDEN)


def _final(sp_w, y_w, degp_w, b):
    return pl.pallas_call(
        _final_body,
        grid=(1,),
        in_specs=[
            pl.BlockSpec((NC, N_WIDE_PAD, 128), lambda i: (0, 0, 0)),
            pl.BlockSpec((N_WIDE, 128), lambda i: (0, 0)),
            pl.BlockSpec((NC, N_WIDE_PAD, 128), lambda i: (0, 0, 0)),
            pl.BlockSpec((1, HIDDEN), lambda i: (0, 0)),
        ],
        out_specs=pl.BlockSpec((N_NODES, HIDDEN), lambda i: (0, 0)),
        out_shape=jax.ShapeDtypeStruct((N_NODES, HIDDEN), jnp.float32),
    )(sp_w, y_w, degp_w, b)


@jax.jit
def kernel(x, edge_index, W1, b1):
    ei = edge_index.astype(jnp.int32).reshape(2, N_CHUNKS, CHUNK)
    x_w = x.reshape(N_WIDE, 8 * D_FEAT)
    xw_w = _matmul(x_w, W1)
    degp = _hist(ei)
    degp_w = degp.reshape(NC, N_WIDE_PAD, 128)
    y_w = _scale(xw_w, degp_w)
    sp = _segsum(ei, y_w.reshape(N_NODES, HIDDEN))
    return _final(sp.reshape(NC, N_WIDE_PAD, 128), y_w, degp_w,
                  b1.reshape(1, HIDDEN))


# speedup vs baseline: 1.0595x; 1.0595x over previous
import functools

import jax
import jax.numpy as jnp
from jax import lax
from jax.experimental import pallas as pl
from jax.experimental.pallas import tpu as pltpu
from jax.experimental.pallas import tpu_sc as plsc

N_NODES = 10000
N_EDGES = 320000
D_FEAT = 128
HIDDEN = 16

NC = 2
NS = 16
LANES = 16

CHUNK = 128
N_CHUNKS = N_EDGES // CHUNK
NB = N_CHUNKS // (NC * NS)
NBUF = 6
NGRP = NB // NBUF
EXTRA_CHUNKS = N_CHUNKS - NB * NC * NS
ROWS_PER_SUB = 624
ROWS_TAIL = N_NODES - NS * ROWS_PER_SUB

N_WIDE = N_NODES * HIDDEN // 128
N_WIDE_PAD = 1256
N_NODES_PAD = N_WIDE_PAD * 128 // HIDDEN

_sc_mesh = plsc.VectorSubcoreMesh(
    core_axis_name="c", subcore_axis_name="s", num_cores=NC, num_subcores=NS
)

_sc_params = pltpu.CompilerParams(use_tc_tiling_on_sc=False)


@functools.partial(
    pl.kernel,
    out_type=jax.ShapeDtypeStruct((NC, N_NODES_PAD, HIDDEN), jnp.float32),
    mesh=_sc_mesh,
    compiler_params=_sc_params,
    scratch_types=[
        pltpu.VMEM((NB + 1, CHUNK), jnp.int32),
        pltpu.VMEM((CHUNK, HIDDEN), jnp.float32),
        pltpu.VMEM((ROWS_PER_SUB, HIDDEN), jnp.float32),
        pltpu.VMEM_SHARED((N_NODES, HIDDEN), jnp.float32),
        pltpu.SemaphoreType.DMA,
    ],
)
def _hist(ei_hbm, out_hbm, idx_v, ones_v, zrows_v, acc_sh, sem):
    c = lax.axis_index("c")
    s = lax.axis_index("s")
    wid = s * NC + c

    @pl.loop(0, CHUNK)
    def _(i):
        ones_v[i, :] = jnp.ones((LANES,), jnp.float32)

    @pl.loop(0, ROWS_PER_SUB)
    def _(i):
        zrows_v[i, :] = jnp.zeros((LANES,), jnp.float32)

    pltpu.sync_copy(ei_hbm.at[1, pl.ds(wid * NB, NB)], idx_v.at[pl.ds(0, NB)])

    @pl.when(wid < EXTRA_CHUNKS)
    def _():
        pltpu.sync_copy(ei_hbm.at[1, pl.ds(NC * NS * NB + wid, 1)],
                        idx_v.at[pl.ds(NB, 1)])

    pltpu.sync_copy(zrows_v, acc_sh.at[pl.ds(s * ROWS_PER_SUB, ROWS_PER_SUB)])

    @pl.when(s == NS - 1)
    def _():
        pltpu.sync_copy(zrows_v.at[pl.ds(0, ROWS_TAIL)],
                        acc_sh.at[pl.ds(NS * ROWS_PER_SUB, ROWS_TAIL)])

    plsc.subcore_barrier()

    @pl.loop(0, NB)
    def _(j):
        pltpu.async_copy(ones_v, acc_sh.at[idx_v.at[j]], sem, add=True)

    @pl.when(wid < EXTRA_CHUNKS)
    def _():
        pltpu.async_copy(ones_v, acc_sh.at[idx_v.at[NB]], sem, add=True)

    @pl.loop(0, NB)
    def _(j):
        pltpu.make_async_copy(ones_v, acc_sh.at[idx_v.at[j]], sem).wait()

    @pl.when(wid < EXTRA_CHUNKS)
    def _():
        pltpu.make_async_copy(ones_v, acc_sh.at[idx_v.at[NB]], sem).wait()

    plsc.subcore_barrier()
    pltpu.sync_copy(
        acc_sh.at[pl.ds(s * ROWS_PER_SUB, ROWS_PER_SUB)],
        out_hbm.at[c, pl.ds(s * ROWS_PER_SUB, ROWS_PER_SUB)],
    )

    @pl.when(s == NS - 1)
    def _():
        pltpu.sync_copy(
            acc_sh.at[pl.ds(NS * ROWS_PER_SUB, ROWS_TAIL)],
            out_hbm.at[c, pl.ds(NS * ROWS_PER_SUB, ROWS_TAIL)],
        )


@functools.partial(
    pl.kernel,
    out_type=jax.ShapeDtypeStruct((NC, N_NODES_PAD, HIDDEN), jnp.float32),
    mesh=_sc_mesh,
    compiler_params=_sc_params,
    scratch_types=[
        pltpu.VMEM((NB + 1, CHUNK), jnp.int32),
        pltpu.VMEM((NB + 1, CHUNK), jnp.int32),
        [pltpu.VMEM((CHUNK, HIDDEN), jnp.float32) for _ in range(NBUF)],
        pltpu.VMEM((ROWS_PER_SUB, HIDDEN), jnp.float32),
        pltpu.VMEM_SHARED((N_NODES, HIDDEN), jnp.float32),
        [pltpu.SemaphoreType.DMA for _ in range(NBUF)],
    ],
)
def _segsum(ei_hbm, y_hbm, out_hbm,
            idx_s_v, idx_d_v, rows_bufs, zrows_v, acc_sh, sems):
    c = lax.axis_index("c")
    s = lax.axis_index("s")
    wid = s * NC + c

    @pl.loop(0, ROWS_PER_SUB)
    def _(i):
        zrows_v[i, :] = jnp.zeros((LANES,), jnp.float32)

    pltpu.sync_copy(ei_hbm.at[0, pl.ds(wid * NB, NB)], idx_s_v.at[pl.ds(0, NB)])
    pltpu.sync_copy(ei_hbm.at[1, pl.ds(wid * NB, NB)], idx_d_v.at[pl.ds(0, NB)])

    @pl.when(wid < EXTRA_CHUNKS)
    def _():
        pltpu.sync_copy(ei_hbm.at[0, pl.ds(NC * NS * NB + wid, 1)],
                        idx_s_v.at[pl.ds(NB, 1)])
        pltpu.sync_copy(ei_hbm.at[1, pl.ds(NC * NS * NB + wid, 1)],
                        idx_d_v.at[pl.ds(NB, 1)])

    rows_slc = pl.ds(s * ROWS_PER_SUB, ROWS_PER_SUB)
    tail_slc = pl.ds(NS * ROWS_PER_SUB, ROWS_TAIL)
    pltpu.sync_copy(zrows_v, acc_sh.at[rows_slc])

    @pl.when(s == NS - 1)
    def _():
        pltpu.sync_copy(zrows_v.at[pl.ds(0, ROWS_TAIL)], acc_sh.at[tail_slc])

    plsc.subcore_barrier()

    def gath(j, buf, sem):
        pltpu.async_copy(y_hbm.at[idx_s_v.at[j]], buf, sem)

    def wait_one(buf, sem):
        pltpu.make_async_copy(y_hbm.at[idx_s_v.at[0]], buf, sem).wait()

    def scat(j, buf, sem):
        pltpu.async_copy(buf, acc_sh.at[idx_d_v.at[j]], sem, add=True)

    for b in range(NBUF):
        gath(b, rows_bufs[b], sems[b])

    @pl.loop(0, NGRP)
    def _(g):
        j0 = g * NBUF
        for b in range(NBUF):
            wait_one(rows_bufs[b], sems[b])
            scat(j0 + b, rows_bufs[b], sems[b])

        @pl.when(g < NGRP - 1)
        def _():
            for b in range(NBUF):
                wait_one(rows_bufs[b], sems[b])
                gath(j0 + NBUF + b, rows_bufs[b], sems[b])

    for b in range(NBUF):
        wait_one(rows_bufs[b], sems[b])

    @pl.when(wid < EXTRA_CHUNKS)
    def _():
        gath(NB, rows_bufs[0], sems[0])
        wait_one(rows_bufs[0], sems[0])
        scat(NB, rows_bufs[0], sems[0])
        wait_one(rows_bufs[0], sems[0])

    plsc.subcore_barrier()
    pltpu.sync_copy(acc_sh.at[rows_slc], out_hbm.at[c, rows_slc])

    @pl.when(s == NS - 1)
    def _():
        pltpu.sync_copy(acc_sh.at[tail_slc], out_hbm.at[c, tail_slc])


def _blockdiag_mask(n_rep, blk_r, blk_c, dtype=jnp.float32):
    shape = (n_rep * blk_r, n_rep * blk_c)
    r = lax.broadcasted_iota(jnp.int32, shape, 0) // blk_r
    c = lax.broadcasted_iota(jnp.int32, shape, 1) // blk_c
    return (r == c).astype(dtype)


def _matmul_body(xw_ref, w_ref, y_ref):
    w = w_ref[...]
    wc = jnp.concatenate([w] * 8, axis=0)
    wt = jnp.concatenate([wc] * 8, axis=1)
    w2 = wt * _blockdiag_mask(8, D_FEAT, HIDDEN)
    y_ref[...] = jnp.dot(xw_ref[...], w2, preferred_element_type=jnp.float32)


def _matmul(x_w, w):
    return pl.pallas_call(
        _matmul_body,
        grid=(1,),
        in_specs=[
            pl.BlockSpec((N_WIDE, 8 * D_FEAT), lambda i: (0, 0)),
            pl.BlockSpec((D_FEAT, HIDDEN), lambda i: (0, 0)),
        ],
        out_specs=pl.BlockSpec((N_WIDE, 128), lambda i: (0, 0)),
        out_shape=jax.ShapeDtypeStruct((N_WIDE, 128), jnp.float32),
    )(x_w, w)


def _scale_body(xw_ref, degp_ref, y_ref):
    degp = degp_ref[:, :N_WIDE, :]
    deg = degp[0] + degp[1] + 1.0
    y_ref[...] = xw_ref[...] * lax.rsqrt(deg)


def _scale(xw_w, degp_w):
    return pl.pallas_call(
        _scale_body,
        grid=(1,),
        in_specs=[
            pl.BlockSpec((N_WIDE, 128), lambda i: (0, 0)),
            pl.BlockSpec((NC, N_WIDE_PAD, 128), lambda i: (0, 0, 0)),
        ],
        out_specs=pl.BlockSpec((N_WIDE, 128), lambda i: (0, 0)),
        out_shape=jax.ShapeDtypeStruct((N_WIDE, 128), jnp.float32),
    )(xw_w, degp_w)


def _final_body(sp_ref, y_ref, degp_ref, b_ref, o_ref):
    degp = degp_ref[:, :N_WIDE, :]
    sp = sp_ref[:, :N_WIDE, :]
    deg = degp[0] + degp[1] + 1.0
    dis = lax.rsqrt(deg)
    b_w = jnp.concatenate([b_ref[...]] * 8, axis=1)
    h = dis * (sp[0] + sp[1] + y_ref[...]) + b_w
    h = jnp.maximum(h, 0.0)
    e = jnp.exp(h)
    g = _blockdiag_mask(8, HIDDEN, HIDDEN)
    s = jnp.dot(e, g, preferred_element_type=jnp.float32)
    o_ref[...] = h - jnp.log(s)


def _final(sp_w, y_w, degp_w, b):
    return pl.pallas_call(
        _final_body,
        grid=(1,),
        in_specs=[
            pl.BlockSpec((NC, N_WIDE_PAD, 128), lambda i: (0, 0, 0)),
            pl.BlockSpec((N_WIDE, 128), lambda i: (0, 0)),
            pl.BlockSpec((NC, N_WIDE_PAD, 128), lambda i: (0, 0, 0)),
            pl.BlockSpec((1, HIDDEN), lambda i: (0, 0)),
        ],
        out_specs=pl.BlockSpec((N_WIDE, 128), lambda i: (0, 0)),
        out_shape=jax.ShapeDtypeStruct((N_WIDE, 128), jnp.float32),
    )(sp_w, y_w, degp_w, b)


@jax.jit
def kernel(x, edge_index, W1, b1):
    ei = edge_index.astype(jnp.int32).reshape(2, N_CHUNKS, CHUNK)
    x_w = x.reshape(N_WIDE, 8 * D_FEAT)
    xw_w = _matmul(x_w, W1)
    degp = _hist(ei)
    degp_w = degp.reshape(NC, N_WIDE_PAD, 128)
    y_w = _scale(xw_w, degp_w)
    sp = _segsum(ei, y_w.reshape(N_NODES, HIDDEN))
    out_w = _final(sp.reshape(NC, N_WIDE_PAD, 128), y_w, degp_w, b1.reshape(1, HIDDEN))
    return out_w.reshape(N_NODES, HIDDEN)


# speedup vs baseline: 1.1012x; 1.0393x over previous
import functools

import jax
import jax.numpy as jnp
from jax import lax
from jax.experimental import pallas as pl
from jax.experimental.pallas import tpu as pltpu
from jax.experimental.pallas import tpu_sc as plsc

N_NODES = 10000
N_EDGES = 320000
D_FEAT = 128
HIDDEN = 16

NC = 2
NS = 16
LANES = 16

CHUNK = 128
N_CHUNKS = N_EDGES // CHUNK
NB = N_CHUNKS // (NC * NS)
NBUF = 13
NGRP = NB // NBUF
EXTRA_CHUNKS = N_CHUNKS - NB * NC * NS
ROWS_PER_SUB = 624
ROWS_TAIL = N_NODES - NS * ROWS_PER_SUB

N_WIDE = N_NODES * HIDDEN // 128
N_WIDE_PAD = 1256
N_NODES_PAD = N_WIDE_PAD * 128 // HIDDEN

_sc_mesh = plsc.VectorSubcoreMesh(
    core_axis_name="c", subcore_axis_name="s", num_cores=NC, num_subcores=NS
)

_sc_params = pltpu.CompilerParams(use_tc_tiling_on_sc=False)


@functools.partial(
    pl.kernel,
    out_type=jax.ShapeDtypeStruct((NC, N_NODES_PAD, HIDDEN), jnp.float32),
    mesh=_sc_mesh,
    compiler_params=_sc_params,
    scratch_types=[
        pltpu.VMEM((NB + 1, CHUNK), jnp.int32),
        pltpu.VMEM((CHUNK, HIDDEN), jnp.float32),
        pltpu.VMEM((ROWS_PER_SUB, HIDDEN), jnp.float32),
        pltpu.VMEM_SHARED((N_NODES, HIDDEN), jnp.float32),
        pltpu.SemaphoreType.DMA,
    ],
)
def _hist(ei_hbm, out_hbm, idx_v, ones_v, zrows_v, acc_sh, sem):
    c = lax.axis_index("c")
    s = lax.axis_index("s")
    wid = s * NC + c

    @pl.loop(0, CHUNK)
    def _(i):
        ones_v[i, :] = jnp.ones((LANES,), jnp.float32)

    @pl.loop(0, ROWS_PER_SUB)
    def _(i):
        zrows_v[i, :] = jnp.zeros((LANES,), jnp.float32)

    pltpu.sync_copy(ei_hbm.at[1, pl.ds(wid * NB, NB)], idx_v.at[pl.ds(0, NB)])

    @pl.when(wid < EXTRA_CHUNKS)
    def _():
        pltpu.sync_copy(ei_hbm.at[1, pl.ds(NC * NS * NB + wid, 1)],
                        idx_v.at[pl.ds(NB, 1)])

    pltpu.sync_copy(zrows_v, acc_sh.at[pl.ds(s * ROWS_PER_SUB, ROWS_PER_SUB)])

    @pl.when(s == NS - 1)
    def _():
        pltpu.sync_copy(zrows_v.at[pl.ds(0, ROWS_TAIL)],
                        acc_sh.at[pl.ds(NS * ROWS_PER_SUB, ROWS_TAIL)])

    plsc.subcore_barrier()

    @pl.loop(0, NB)
    def _(j):
        pltpu.async_copy(ones_v, acc_sh.at[idx_v.at[j]], sem, add=True)

    @pl.when(wid < EXTRA_CHUNKS)
    def _():
        pltpu.async_copy(ones_v, acc_sh.at[idx_v.at[NB]], sem, add=True)

    @pl.loop(0, NB)
    def _(j):
        pltpu.make_async_copy(ones_v, acc_sh.at[idx_v.at[j]], sem).wait()

    @pl.when(wid < EXTRA_CHUNKS)
    def _():
        pltpu.make_async_copy(ones_v, acc_sh.at[idx_v.at[NB]], sem).wait()

    plsc.subcore_barrier()
    pltpu.sync_copy(
        acc_sh.at[pl.ds(s * ROWS_PER_SUB, ROWS_PER_SUB)],
        out_hbm.at[c, pl.ds(s * ROWS_PER_SUB, ROWS_PER_SUB)],
    )

    @pl.when(s == NS - 1)
    def _():
        pltpu.sync_copy(
            acc_sh.at[pl.ds(NS * ROWS_PER_SUB, ROWS_TAIL)],
            out_hbm.at[c, pl.ds(NS * ROWS_PER_SUB, ROWS_TAIL)],
        )


@functools.partial(
    pl.kernel,
    out_type=jax.ShapeDtypeStruct((NC, N_NODES_PAD, HIDDEN), jnp.float32),
    mesh=_sc_mesh,
    compiler_params=_sc_params,
    scratch_types=[
        pltpu.VMEM((NB + 1, CHUNK), jnp.int32),
        pltpu.VMEM((NB + 1, CHUNK), jnp.int32),
        [pltpu.VMEM((CHUNK, HIDDEN), jnp.float32) for _ in range(NBUF)],
        pltpu.VMEM((ROWS_PER_SUB, HIDDEN), jnp.float32),
        pltpu.VMEM_SHARED((N_NODES, HIDDEN), jnp.float32),
        [pltpu.SemaphoreType.DMA for _ in range(NBUF)],
    ],
)
def _segsum(ei_hbm, y_hbm, out_hbm,
            idx_s_v, idx_d_v, rows_bufs, zrows_v, acc_sh, sems):
    c = lax.axis_index("c")
    s = lax.axis_index("s")
    wid = s * NC + c

    @pl.loop(0, ROWS_PER_SUB)
    def _(i):
        zrows_v[i, :] = jnp.zeros((LANES,), jnp.float32)

    pltpu.sync_copy(ei_hbm.at[0, pl.ds(wid * NB, NB)], idx_s_v.at[pl.ds(0, NB)])
    pltpu.sync_copy(ei_hbm.at[1, pl.ds(wid * NB, NB)], idx_d_v.at[pl.ds(0, NB)])

    @pl.when(wid < EXTRA_CHUNKS)
    def _():
        pltpu.sync_copy(ei_hbm.at[0, pl.ds(NC * NS * NB + wid, 1)],
                        idx_s_v.at[pl.ds(NB, 1)])
        pltpu.sync_copy(ei_hbm.at[1, pl.ds(NC * NS * NB + wid, 1)],
                        idx_d_v.at[pl.ds(NB, 1)])

    rows_slc = pl.ds(s * ROWS_PER_SUB, ROWS_PER_SUB)
    tail_slc = pl.ds(NS * ROWS_PER_SUB, ROWS_TAIL)
    pltpu.sync_copy(zrows_v, acc_sh.at[rows_slc])

    @pl.when(s == NS - 1)
    def _():
        pltpu.sync_copy(zrows_v.at[pl.ds(0, ROWS_TAIL)], acc_sh.at[tail_slc])

    plsc.subcore_barrier()

    def gath(j, buf, sem):
        pltpu.async_copy(y_hbm.at[idx_s_v.at[j]], buf, sem)

    def wait_one(buf, sem):
        pltpu.make_async_copy(y_hbm.at[idx_s_v.at[0]], buf, sem).wait()

    def scat(j, buf, sem):
        pltpu.async_copy(buf, acc_sh.at[idx_d_v.at[j]], sem, add=True)

    for b in range(NBUF):
        gath(b, rows_bufs[b], sems[b])

    @pl.loop(0, NGRP)
    def _(g):
        j0 = g * NBUF
        for b in range(NBUF):
            wait_one(rows_bufs[b], sems[b])
            scat(j0 + b, rows_bufs[b], sems[b])

        @pl.when(g < NGRP - 1)
        def _():
            for b in range(NBUF):
                wait_one(rows_bufs[b], sems[b])
                gath(j0 + NBUF + b, rows_bufs[b], sems[b])

    for b in range(NBUF):
        wait_one(rows_bufs[b], sems[b])

    @pl.when(wid < EXTRA_CHUNKS)
    def _():
        gath(NB, rows_bufs[0], sems[0])
        wait_one(rows_bufs[0], sems[0])
        scat(NB, rows_bufs[0], sems[0])
        wait_one(rows_bufs[0], sems[0])

    plsc.subcore_barrier()
    pltpu.sync_copy(acc_sh.at[rows_slc], out_hbm.at[c, rows_slc])

    @pl.when(s == NS - 1)
    def _():
        pltpu.sync_copy(acc_sh.at[tail_slc], out_hbm.at[c, tail_slc])


def _blockdiag_mask(n_rep, blk_r, blk_c, dtype=jnp.float32):
    shape = (n_rep * blk_r, n_rep * blk_c)
    r = lax.broadcasted_iota(jnp.int32, shape, 0) // blk_r
    c = lax.broadcasted_iota(jnp.int32, shape, 1) // blk_c
    return (r == c).astype(dtype)


def _matmul_body(xw_ref, w_ref, y_ref):
    w = w_ref[...]
    wc = jnp.concatenate([w] * 8, axis=0)
    wt = jnp.concatenate([wc] * 8, axis=1)
    w2 = wt * _blockdiag_mask(8, D_FEAT, HIDDEN)
    y_ref[...] = jnp.dot(xw_ref[...], w2, preferred_element_type=jnp.float32)


def _matmul(x_w, w):
    return pl.pallas_call(
        _matmul_body,
        grid=(1,),
        in_specs=[
            pl.BlockSpec((N_WIDE, 8 * D_FEAT), lambda i: (0, 0)),
            pl.BlockSpec((D_FEAT, HIDDEN), lambda i: (0, 0)),
        ],
        out_specs=pl.BlockSpec((N_WIDE, 128), lambda i: (0, 0)),
        out_shape=jax.ShapeDtypeStruct((N_WIDE, 128), jnp.float32),
    )(x_w, w)


def _scale_body(xw_ref, degp_ref, y_ref):
    degp = degp_ref[:, :N_WIDE, :]
    deg = degp[0] + degp[1] + 1.0
    y_ref[...] = xw_ref[...] * lax.rsqrt(deg)


def _scale(xw_w, degp_w):
    return pl.pallas_call(
        _scale_body,
        grid=(1,),
        in_specs=[
            pl.BlockSpec((N_WIDE, 128), lambda i: (0, 0)),
            pl.BlockSpec((NC, N_WIDE_PAD, 128), lambda i: (0, 0, 0)),
        ],
        out_specs=pl.BlockSpec((N_WIDE, 128), lambda i: (0, 0)),
        out_shape=jax.ShapeDtypeStruct((N_WIDE, 128), jnp.float32),
    )(xw_w, degp_w)


def _final_body(sp_ref, y_ref, degp_ref, b_ref, o_ref):
    degp = degp_ref[:, :N_WIDE, :]
    sp = sp_ref[:, :N_WIDE, :]
    deg = degp[0] + degp[1] + 1.0
    dis = lax.rsqrt(deg)
    b_w = jnp.concatenate([b_ref[...]] * 8, axis=1)
    h = dis * (sp[0] + sp[1] + y_ref[...]) + b_w
    h = jnp.maximum(h, 0.0)
    e = jnp.exp(h)
    g = _blockdiag_mask(8, HIDDEN, HIDDEN)
    s = jnp.dot(e, g, preferred_element_type=jnp.float32)
    o_ref[...] = h - jnp.log(s)


def _final(sp_w, y_w, degp_w, b):
    return pl.pallas_call(
        _final_body,
        grid=(1,),
        in_specs=[
            pl.BlockSpec((NC, N_WIDE_PAD, 128), lambda i: (0, 0, 0)),
            pl.BlockSpec((N_WIDE, 128), lambda i: (0, 0)),
            pl.BlockSpec((NC, N_WIDE_PAD, 128), lambda i: (0, 0, 0)),
            pl.BlockSpec((1, HIDDEN), lambda i: (0, 0)),
        ],
        out_specs=pl.BlockSpec((N_WIDE, 128), lambda i: (0, 0)),
        out_shape=jax.ShapeDtypeStruct((N_WIDE, 128), jnp.float32),
    )(sp_w, y_w, degp_w, b)


@jax.jit
def kernel(x, edge_index, W1, b1):
    ei = edge_index.astype(jnp.int32).reshape(2, N_CHUNKS, CHUNK)
    x_w = x.reshape(N_WIDE, 8 * D_FEAT)
    xw_w = _matmul(x_w, W1)
    degp = _hist(ei)
    degp_w = degp.reshape(NC, N_WIDE_PAD, 128)
    y_w = _scale(xw_w, degp_w)
    sp = _segsum(ei, y_w.reshape(N_NODES, HIDDEN))
    out_w = _final(sp.reshape(NC, N_WIDE_PAD, 128), y_w, degp_w, b1.reshape(1, HIDDEN))
    return out_w.reshape(N_NODES, HIDDEN)


# speedup vs baseline: 1.1110x; 1.0089x over previous
import functools

import jax
import jax.numpy as jnp
from jax import lax
from jax.experimental import pallas as pl
from jax.experimental.pallas import tpu as pltpu
from jax.experimental.pallas import tpu_sc as plsc

N_NODES = 10000
N_EDGES = 320000
D_FEAT = 128
HIDDEN = 16

NC = 2
NS = 16
LANES = 16

CHUNK = 128
N_CHUNKS = N_EDGES // CHUNK
NB = N_CHUNKS // (NC * NS)
NBUF = 26
NGRP = NB // NBUF
EXTRA_CHUNKS = N_CHUNKS - NB * NC * NS
ROWS_PER_SUB = 624
ROWS_TAIL = N_NODES - NS * ROWS_PER_SUB

N_WIDE = N_NODES * HIDDEN // 128
N_WIDE_PAD = 1256
N_NODES_PAD = N_WIDE_PAD * 128 // HIDDEN

_sc_mesh = plsc.VectorSubcoreMesh(
    core_axis_name="c", subcore_axis_name="s", num_cores=NC, num_subcores=NS
)

_sc_params = pltpu.CompilerParams(use_tc_tiling_on_sc=False)


@functools.partial(
    pl.kernel,
    out_type=jax.ShapeDtypeStruct((NC, N_NODES_PAD, HIDDEN), jnp.float32),
    mesh=_sc_mesh,
    compiler_params=_sc_params,
    scratch_types=[
        pltpu.VMEM((NB + 1, CHUNK), jnp.int32),
        pltpu.VMEM((CHUNK, HIDDEN), jnp.float32),
        pltpu.VMEM((ROWS_PER_SUB, HIDDEN), jnp.float32),
        pltpu.VMEM_SHARED((N_NODES, HIDDEN), jnp.float32),
        pltpu.SemaphoreType.DMA,
    ],
)
def _hist(ei_hbm, out_hbm, idx_v, ones_v, zrows_v, acc_sh, sem):
    c = lax.axis_index("c")
    s = lax.axis_index("s")
    wid = s * NC + c

    @pl.loop(0, CHUNK)
    def _(i):
        ones_v[i, :] = jnp.ones((LANES,), jnp.float32)

    @pl.loop(0, ROWS_PER_SUB)
    def _(i):
        zrows_v[i, :] = jnp.zeros((LANES,), jnp.float32)

    pltpu.sync_copy(ei_hbm.at[1, pl.ds(wid * NB, NB)], idx_v.at[pl.ds(0, NB)])

    @pl.when(wid < EXTRA_CHUNKS)
    def _():
        pltpu.sync_copy(ei_hbm.at[1, pl.ds(NC * NS * NB + wid, 1)],
                        idx_v.at[pl.ds(NB, 1)])

    pltpu.sync_copy(zrows_v, acc_sh.at[pl.ds(s * ROWS_PER_SUB, ROWS_PER_SUB)])

    @pl.when(s == NS - 1)
    def _():
        pltpu.sync_copy(zrows_v.at[pl.ds(0, ROWS_TAIL)],
                        acc_sh.at[pl.ds(NS * ROWS_PER_SUB, ROWS_TAIL)])

    plsc.subcore_barrier()

    @pl.loop(0, NB)
    def _(j):
        pltpu.async_copy(ones_v, acc_sh.at[idx_v.at[j]], sem, add=True)

    @pl.when(wid < EXTRA_CHUNKS)
    def _():
        pltpu.async_copy(ones_v, acc_sh.at[idx_v.at[NB]], sem, add=True)

    @pl.loop(0, NB)
    def _(j):
        pltpu.make_async_copy(ones_v, acc_sh.at[idx_v.at[j]], sem).wait()

    @pl.when(wid < EXTRA_CHUNKS)
    def _():
        pltpu.make_async_copy(ones_v, acc_sh.at[idx_v.at[NB]], sem).wait()

    plsc.subcore_barrier()
    pltpu.sync_copy(
        acc_sh.at[pl.ds(s * ROWS_PER_SUB, ROWS_PER_SUB)],
        out_hbm.at[c, pl.ds(s * ROWS_PER_SUB, ROWS_PER_SUB)],
    )

    @pl.when(s == NS - 1)
    def _():
        pltpu.sync_copy(
            acc_sh.at[pl.ds(NS * ROWS_PER_SUB, ROWS_TAIL)],
            out_hbm.at[c, pl.ds(NS * ROWS_PER_SUB, ROWS_TAIL)],
        )


@functools.partial(
    pl.kernel,
    out_type=jax.ShapeDtypeStruct((NC, N_NODES_PAD, HIDDEN), jnp.float32),
    mesh=_sc_mesh,
    compiler_params=_sc_params,
    scratch_types=[
        pltpu.VMEM((NB + 1, CHUNK), jnp.int32),
        pltpu.VMEM((NB + 1, CHUNK), jnp.int32),
        [pltpu.VMEM((CHUNK, HIDDEN), jnp.float32) for _ in range(NBUF)],
        pltpu.VMEM((ROWS_PER_SUB, HIDDEN), jnp.float32),
        pltpu.VMEM_SHARED((N_NODES, HIDDEN), jnp.float32),
        [pltpu.SemaphoreType.DMA for _ in range(NBUF)],
    ],
)
def _segsum(ei_hbm, y_hbm, out_hbm,
            idx_s_v, idx_d_v, rows_bufs, zrows_v, acc_sh, sems):
    c = lax.axis_index("c")
    s = lax.axis_index("s")
    wid = s * NC + c

    @pl.loop(0, ROWS_PER_SUB)
    def _(i):
        zrows_v[i, :] = jnp.zeros((LANES,), jnp.float32)

    pltpu.sync_copy(ei_hbm.at[0, pl.ds(wid * NB, NB)], idx_s_v.at[pl.ds(0, NB)])
    pltpu.sync_copy(ei_hbm.at[1, pl.ds(wid * NB, NB)], idx_d_v.at[pl.ds(0, NB)])

    @pl.when(wid < EXTRA_CHUNKS)
    def _():
        pltpu.sync_copy(ei_hbm.at[0, pl.ds(NC * NS * NB + wid, 1)],
                        idx_s_v.at[pl.ds(NB, 1)])
        pltpu.sync_copy(ei_hbm.at[1, pl.ds(NC * NS * NB + wid, 1)],
                        idx_d_v.at[pl.ds(NB, 1)])

    rows_slc = pl.ds(s * ROWS_PER_SUB, ROWS_PER_SUB)
    tail_slc = pl.ds(NS * ROWS_PER_SUB, ROWS_TAIL)
    pltpu.sync_copy(zrows_v, acc_sh.at[rows_slc])

    @pl.when(s == NS - 1)
    def _():
        pltpu.sync_copy(zrows_v.at[pl.ds(0, ROWS_TAIL)], acc_sh.at[tail_slc])

    plsc.subcore_barrier()

    def gath(j, buf, sem):
        pltpu.async_copy(y_hbm.at[idx_s_v.at[j]], buf, sem)

    def wait_one(buf, sem):
        pltpu.make_async_copy(y_hbm.at[idx_s_v.at[0]], buf, sem).wait()

    def scat(j, buf, sem):
        pltpu.async_copy(buf, acc_sh.at[idx_d_v.at[j]], sem, add=True)

    for b in range(NBUF):
        gath(b, rows_bufs[b], sems[b])

    @pl.loop(0, NGRP)
    def _(g):
        j0 = g * NBUF
        for b in range(NBUF):
            wait_one(rows_bufs[b], sems[b])
            scat(j0 + b, rows_bufs[b], sems[b])

        @pl.when(g < NGRP - 1)
        def _():
            for b in range(NBUF):
                wait_one(rows_bufs[b], sems[b])
                gath(j0 + NBUF + b, rows_bufs[b], sems[b])

    for b in range(NBUF):
        wait_one(rows_bufs[b], sems[b])

    @pl.when(wid < EXTRA_CHUNKS)
    def _():
        gath(NB, rows_bufs[0], sems[0])
        wait_one(rows_bufs[0], sems[0])
        scat(NB, rows_bufs[0], sems[0])
        wait_one(rows_bufs[0], sems[0])

    plsc.subcore_barrier()
    pltpu.sync_copy(acc_sh.at[rows_slc], out_hbm.at[c, rows_slc])

    @pl.when(s == NS - 1)
    def _():
        pltpu.sync_copy(acc_sh.at[tail_slc], out_hbm.at[c, tail_slc])


def _blockdiag_mask(n_rep, blk_r, blk_c, dtype=jnp.float32):
    shape = (n_rep * blk_r, n_rep * blk_c)
    r = lax.broadcasted_iota(jnp.int32, shape, 0) // blk_r
    c = lax.broadcasted_iota(jnp.int32, shape, 1) // blk_c
    return (r == c).astype(dtype)


def _matmul_body(xw_ref, w_ref, y_ref):
    w = w_ref[...]
    wc = jnp.concatenate([w] * 8, axis=0)
    wt = jnp.concatenate([wc] * 8, axis=1)
    w2 = wt * _blockdiag_mask(8, D_FEAT, HIDDEN)
    y_ref[...] = jnp.dot(xw_ref[...], w2, preferred_element_type=jnp.float32)


def _matmul(x_w, w):
    return pl.pallas_call(
        _matmul_body,
        grid=(1,),
        in_specs=[
            pl.BlockSpec((N_WIDE, 8 * D_FEAT), lambda i: (0, 0)),
            pl.BlockSpec((D_FEAT, HIDDEN), lambda i: (0, 0)),
        ],
        out_specs=pl.BlockSpec((N_WIDE, 128), lambda i: (0, 0)),
        out_shape=jax.ShapeDtypeStruct((N_WIDE, 128), jnp.float32),
    )(x_w, w)


def _scale_body(xw_ref, degp_ref, y_ref):
    degp = degp_ref[:, :N_WIDE, :]
    deg = degp[0] + degp[1] + 1.0
    y_ref[...] = xw_ref[...] * lax.rsqrt(deg)


def _scale(xw_w, degp_w):
    return pl.pallas_call(
        _scale_body,
        grid=(1,),
        in_specs=[
            pl.BlockSpec((N_WIDE, 128), lambda i: (0, 0)),
            pl.BlockSpec((NC, N_WIDE_PAD, 128), lambda i: (0, 0, 0)),
        ],
        out_specs=pl.BlockSpec((N_WIDE, 128), lambda i: (0, 0)),
        out_shape=jax.ShapeDtypeStruct((N_WIDE, 128), jnp.float32),
    )(xw_w, degp_w)


def _final_body(sp_ref, y_ref, degp_ref, b_ref, o_ref):
    degp = degp_ref[:, :N_WIDE, :]
    sp = sp_ref[:, :N_WIDE, :]
    deg = degp[0] + degp[1] + 1.0
    dis = lax.rsqrt(deg)
    b_w = jnp.concatenate([b_ref[...]] * 8, axis=1)
    h = dis * (sp[0] + sp[1] + y_ref[...]) + b_w
    h = jnp.maximum(h, 0.0)
    e = jnp.exp(h)
    g = _blockdiag_mask(8, HIDDEN, HIDDEN)
    s = jnp.dot(e, g, preferred_element_type=jnp.float32)
    o_ref[...] = h - jnp.log(s)


def _final(sp_w, y_w, degp_w, b):
    return pl.pallas_call(
        _final_body,
        grid=(1,),
        in_specs=[
            pl.BlockSpec((NC, N_WIDE_PAD, 128), lambda i: (0, 0, 0)),
            pl.BlockSpec((N_WIDE, 128), lambda i: (0, 0)),
            pl.BlockSpec((NC, N_WIDE_PAD, 128), lambda i: (0, 0, 0)),
            pl.BlockSpec((1, HIDDEN), lambda i: (0, 0)),
        ],
        out_specs=pl.BlockSpec((N_WIDE, 128), lambda i: (0, 0)),
        out_shape=jax.ShapeDtypeStruct((N_WIDE, 128), jnp.float32),
    )(sp_w, y_w, degp_w, b)


@jax.jit
def kernel(x, edge_index, W1, b1):
    ei = edge_index.astype(jnp.int32).reshape(2, N_CHUNKS, CHUNK)
    x_w = x.reshape(N_WIDE, 8 * D_FEAT)
    xw_w = _matmul(x_w, W1)
    degp = _hist(ei)
    degp_w = degp.reshape(NC, N_WIDE_PAD, 128)
    y_w = _scale(xw_w, degp_w)
    sp = _segsum(ei, y_w.reshape(N_NODES, HIDDEN))
    out_w = _final(sp.reshape(NC, N_WIDE_PAD, 128), y_w, degp_w, b1.reshape(1, HIDDEN))
    return out_w.reshape(N_NODES, HIDDEN)
